# baseline (speedup 1.0000x reference)
import jax
import jax.numpy as jnp
from jax.experimental import pallas as pl

_A = 8
_NA = 16
_D = 128
_OUT = 64


def _body(h_ref, pi_ref, act_ref, obs_ref,
          kW1_ref, kb1_ref, kW2_ref, kb2_ref,
          qW1_ref, qb1_ref, qW2_ref, qb2_ref,
          out_ref, w_ref):
    EB = h_ref.shape[0]
    R = EB * _A
    hb = h_ref[...].reshape(R, _D)
    key = jnp.tanh(
        jnp.dot(hb, kW1_ref[...], preferred_element_type=jnp.float32)
        + kb1_ref[...])
    key = (jnp.dot(key, kW2_ref[...], preferred_element_type=jnp.float32)
           + kb2_ref[...]).reshape(EB, _A, _OUT)
    qry = jnp.tanh(
        jnp.dot(hb, qW1_ref[...], preferred_element_type=jnp.float32)
        + qb1_ref[...])
    qry = (jnp.dot(qry, qW2_ref[...], preferred_element_type=jnp.float32)
           + qb2_ref[...]).reshape(EB, _A, _OUT)
    s = jnp.sum(qry[:, :, None, :] * key[:, None, :, :], axis=-1)
    w = jax.nn.sigmoid(s * 0.125)
    pi = pi_ref[...]
    act = act_ref[...]
    pib = pi[:, None, :, :]
    z = w[..., None] * (act[:, None, :, :] - pib) + pib
    S = jnp.sum(z, axis=2)
    zz = (S[:, :, None, :] - z + pib) * 0.125
    obs_big = jnp.broadcast_to(obs_ref[...][:, None, :, :],
                               (EB, _A, _A, _D)).reshape(R, _A, _D)
    out_ref[:, :, 0:_D] = obs_big
    out_ref[:, :, _D:] = zz.reshape(R, _A, _NA)
    w_ref[...] = w.reshape(R, _A)[..., None]


def kernel(h, policies, actions, obs_proc, edge_index,
           kW1, kb1, kW2, kb2, qW1, qb1, qW2, qb2):
    N = h.shape[0]
    E = N // _A
    EB = 16
    grid = (E // EB,)
    h3 = h.reshape(E, _A, _D)
    pi3 = policies.reshape(E, _A, _NA)
    act3 = actions.reshape(E, _A, _NA)
    obs3 = obs_proc.reshape(E, _A, _D)

    def blk(shape):
        return pl.BlockSpec(shape, lambda b: (b,) + (0,) * (len(shape) - 1))

    def full(shape):
        return pl.BlockSpec(shape, lambda b: (0,) * len(shape))

    out, w = pl.pallas_call(
        _body,
        grid=grid,
        in_specs=[
            blk((EB, _A, _D)),
            blk((EB, _A, _NA)),
            blk((EB, _A, _NA)),
            blk((EB, _A, _D)),
            full((_D, 32)), full((1, 32)), full((32, _OUT)), full((1, _OUT)),
            full((_D, 32)), full((1, 32)), full((32, _OUT)), full((1, _OUT)),
        ],
        out_specs=[
            blk((EB * _A, _A, _D + _NA)),
            blk((EB * _A, _A, 1)),
        ],
        out_shape=[
            jax.ShapeDtypeStruct((N, _A, _D + _NA), jnp.float32),
            jax.ShapeDtypeStruct((N, _A, 1), jnp.float32),
        ],
    )(h3, pi3, act3, obs3,
      kW1, kb1.reshape(1, 32), kW2, kb2.reshape(1, _OUT),
      qW1, qb1.reshape(1, 32), qW2, qb2.reshape(1, _OUT))
    return out, w


# speedup vs baseline: 1.4916x; 1.4916x over previous
import jax
import jax.numpy as jnp
from jax.experimental import pallas as pl

_A = 8
_NA = 16
_D = 128
_OUT = 64


def _body(h_ref, pi_ref, act_ref, obs_ref,
          kW1_ref, kb1_ref, kW2_ref, kb2_ref,
          qW1_ref, qb1_ref, qW2_ref, qb2_ref,
          out_ref, w_ref):
    EB = h_ref.shape[0]
    R = EB * _A
    hb = h_ref[...].reshape(R, _D)
    key = jnp.tanh(
        jnp.dot(hb, kW1_ref[...], preferred_element_type=jnp.float32)
        + kb1_ref[...])
    key = (jnp.dot(key, kW2_ref[...], preferred_element_type=jnp.float32)
           + kb2_ref[...]).reshape(EB, _A, _OUT)
    qry = jnp.tanh(
        jnp.dot(hb, qW1_ref[...], preferred_element_type=jnp.float32)
        + qb1_ref[...])
    qry = (jnp.dot(qry, qW2_ref[...], preferred_element_type=jnp.float32)
           + qb2_ref[...]).reshape(EB, _A, _OUT)
    s = jnp.sum(qry[:, :, None, :] * key[:, None, :, :], axis=-1)
    w = jax.nn.sigmoid(s * 0.125)
    pi = pi_ref[...]
    act = act_ref[...]
    pib = pi[:, None, :, :]
    z = w[..., None] * (act[:, None, :, :] - pib) + pib
    S = jnp.sum(z, axis=2)
    zz = (S[:, :, None, :] - z + pib) * 0.125
    obs = obs_ref[...]
    for i in range(_A):
        out_ref[:, i, :, 0:_D] = obs
    out_ref[:, :, :, _D:] = zz
    w_ref[...] = w


def kernel(h, policies, actions, obs_proc, edge_index,
           kW1, kb1, kW2, kb2, qW1, qb1, qW2, qb2):
    N = h.shape[0]
    E = N // _A
    EB = 32
    grid = (E // EB,)
    h3 = h.reshape(E, _A, _D)
    pi3 = policies.reshape(E, _A, _NA)
    act3 = actions.reshape(E, _A, _NA)
    obs3 = obs_proc.reshape(E, _A, _D)

    def blk(shape):
        return pl.BlockSpec(shape, lambda b: (b,) + (0,) * (len(shape) - 1))

    def full(shape):
        return pl.BlockSpec(shape, lambda b: (0,) * len(shape))

    out, w = pl.pallas_call(
        _body,
        grid=grid,
        in_specs=[
            blk((EB, _A, _D)),
            blk((EB, _A, _NA)),
            blk((EB, _A, _NA)),
            blk((EB, _A, _D)),
            full((_D, 32)), full((1, 32)), full((32, _OUT)), full((1, _OUT)),
            full((_D, 32)), full((1, 32)), full((32, _OUT)), full((1, _OUT)),
        ],
        out_specs=[
            blk((EB, _A, _A, _D + _NA)),
            blk((EB, _A, _A)),
        ],
        out_shape=[
            jax.ShapeDtypeStruct((E, _A, _A, _D + _NA), jnp.float32),
            jax.ShapeDtypeStruct((E, _A, _A), jnp.float32),
        ],
    )(h3, pi3, act3, obs3,
      kW1, kb1.reshape(1, 32), kW2, kb2.reshape(1, _OUT),
      qW1, qb1.reshape(1, 32), qW2, qb2.reshape(1, _OUT))
    return out.reshape(N, _A, _D + _NA), w.reshape(N, _A, 1)


# speedup vs baseline: 1.6700x; 1.1196x over previous
import jax
import jax.numpy as jnp
from jax.experimental import pallas as pl

_A = 8
_NA = 16
_D = 128
_OUT = 64


def _body(h_ref, pi_ref, act_ref, obs_ref,
          kW1_ref, kb1_ref, kW2_ref, kb2_ref,
          qW1_ref, qb1_ref, qW2_ref, qb2_ref,
          out_ref, w_ref):
    EB = h_ref.shape[0]
    R = EB * _A
    hb = h_ref[...].reshape(R, _D)
    key = jnp.tanh(
        jnp.dot(hb, kW1_ref[...], preferred_element_type=jnp.float32)
        + kb1_ref[...])
    key = (jnp.dot(key, kW2_ref[...], preferred_element_type=jnp.float32)
           + kb2_ref[...]).reshape(EB, _A, _OUT)
    qry = jnp.tanh(
        jnp.dot(hb, qW1_ref[...], preferred_element_type=jnp.float32)
        + qb1_ref[...])
    qry = (jnp.dot(qry, qW2_ref[...], preferred_element_type=jnp.float32)
           + qb2_ref[...]).reshape(EB, _A, _OUT)
    s = jnp.sum(qry[:, :, None, :] * key[:, None, :, :], axis=-1)
    w = jax.nn.sigmoid(s * 0.125)
    pi = pi_ref[...]
    act = act_ref[...]
    pib = pi[:, None, :, :]
    z = w[..., None] * (act[:, None, :, :] - pib) + pib
    S = jnp.sum(z, axis=2)
    zz = (S[:, :, None, :] - z + pib) * 0.125
    obs = obs_ref[...]
    for i in range(_A):
        out_ref[:, i, :, 0:_D] = obs
    out_ref[:, :, :, _D:] = zz
    w_ref[...] = w


def kernel(h, policies, actions, obs_proc, edge_index,
           kW1, kb1, kW2, kb2, qW1, qb1, qW2, qb2):
    N = h.shape[0]
    E = N // _A
    EB = 64
    grid = (E // EB,)
    h3 = h.reshape(E, _A, _D)
    pi3 = policies.reshape(E, _A, _NA)
    act3 = actions.reshape(E, _A, _NA)
    obs3 = obs_proc.reshape(E, _A, _D)

    def blk(shape):
        return pl.BlockSpec(shape, lambda b: (b,) + (0,) * (len(shape) - 1))

    def full(shape):
        return pl.BlockSpec(shape, lambda b: (0,) * len(shape))

    out, w = pl.pallas_call(
        _body,
        grid=grid,
        in_specs=[
            blk((EB, _A, _D)),
            blk((EB, _A, _NA)),
            blk((EB, _A, _NA)),
            blk((EB, _A, _D)),
            full((_D, 32)), full((1, 32)), full((32, _OUT)), full((1, _OUT)),
            full((_D, 32)), full((1, 32)), full((32, _OUT)), full((1, _OUT)),
        ],
        out_specs=[
            blk((EB, _A, _A, _D + _NA)),
            blk((EB, _A, _A)),
        ],
        out_shape=[
            jax.ShapeDtypeStruct((E, _A, _A, _D + _NA), jnp.float32),
            jax.ShapeDtypeStruct((E, _A, _A), jnp.float32),
        ],
    )(h3, pi3, act3, obs3,
      kW1, kb1.reshape(1, 32), kW2, kb2.reshape(1, _OUT),
      qW1, qb1.reshape(1, 32), qW2, qb2.reshape(1, _OUT))
    return out.reshape(N, _A, _D + _NA), w.reshape(N, _A, 1)


# speedup vs baseline: 1.7571x; 1.0522x over previous
import jax
import jax.numpy as jnp
from jax.experimental import pallas as pl

_A = 8
_NA = 16
_D = 128
_OUT = 64


def _body(h_ref, pi_ref, act_ref, obs_ref,
          kW1_ref, kb1_ref, kW2_ref, kb2_ref,
          qW1_ref, qb1_ref, qW2_ref, qb2_ref,
          out_ref, w_ref):
    EB = h_ref.shape[0]
    R = EB * _A
    hb = h_ref[...].reshape(R, _D)
    key = jnp.tanh(
        jnp.dot(hb, kW1_ref[...], preferred_element_type=jnp.float32)
        + kb1_ref[...])
    key = (jnp.dot(key, kW2_ref[...], preferred_element_type=jnp.float32)
           + kb2_ref[...]).reshape(EB, _A, _OUT)
    qry = jnp.tanh(
        jnp.dot(hb, qW1_ref[...], preferred_element_type=jnp.float32)
        + qb1_ref[...])
    qry = (jnp.dot(qry, qW2_ref[...], preferred_element_type=jnp.float32)
           + qb2_ref[...]).reshape(EB, _A, _OUT)
    s = jnp.sum(qry[:, :, None, :] * key[:, None, :, :], axis=-1)
    w = jax.nn.sigmoid(s * 0.125)
    pi = pi_ref[...]
    act = act_ref[...]
    pib = pi[:, None, :, :]
    z = w[..., None] * (act[:, None, :, :] - pib) + pib
    S = jnp.sum(z, axis=2)
    zz = (S[:, :, None, :] - z + pib) * 0.125
    obs = obs_ref[...]
    for i in range(_A):
        out_ref[:, i, :, 0:_D] = obs
    out_ref[:, :, :, _D:] = zz
    w_ref[...] = w


def kernel(h, policies, actions, obs_proc, edge_index,
           kW1, kb1, kW2, kb2, qW1, qb1, qW2, qb2):
    N = h.shape[0]
    E = N // _A
    EB = 128
    grid = (E // EB,)
    h3 = h.reshape(E, _A, _D)
    pi3 = policies.reshape(E, _A, _NA)
    act3 = actions.reshape(E, _A, _NA)
    obs3 = obs_proc.reshape(E, _A, _D)

    def blk(shape):
        return pl.BlockSpec(shape, lambda b: (b,) + (0,) * (len(shape) - 1))

    def full(shape):
        return pl.BlockSpec(shape, lambda b: (0,) * len(shape))

    out, w = pl.pallas_call(
        _body,
        grid=grid,
        in_specs=[
            blk((EB, _A, _D)),
            blk((EB, _A, _NA)),
            blk((EB, _A, _NA)),
            blk((EB, _A, _D)),
            full((_D, 32)), full((1, 32)), full((32, _OUT)), full((1, _OUT)),
            full((_D, 32)), full((1, 32)), full((32, _OUT)), full((1, _OUT)),
        ],
        out_specs=[
            blk((EB, _A, _A, _D + _NA)),
            blk((EB, _A, _A)),
        ],
        out_shape=[
            jax.ShapeDtypeStruct((E, _A, _A, _D + _NA), jnp.float32),
            jax.ShapeDtypeStruct((E, _A, _A), jnp.float32),
        ],
    )(h3, pi3, act3, obs3,
      kW1, kb1.reshape(1, 32), kW2, kb2.reshape(1, _OUT),
      qW1, qb1.reshape(1, 32), qW2, qb2.reshape(1, _OUT))
    return out.reshape(N, _A, _D + _NA), w.reshape(N, _A, 1)


# speedup vs baseline: 1.7720x; 1.0085x over previous
import jax
import jax.numpy as jnp
from jax.experimental import pallas as pl

_A = 8
_NA = 16
_D = 128
_OUT = 64


def _body(h_ref, pi_ref, act_ref, obs_ref,
          kW1_ref, kb1_ref, kW2_ref, kb2_ref,
          qW1_ref, qb1_ref, qW2_ref, qb2_ref,
          out_ref, w_ref):
    EB = h_ref.shape[0]
    R = EB * _A
    hb = h_ref[...].reshape(R, _D)
    key = jnp.tanh(
        jnp.dot(hb, kW1_ref[...], preferred_element_type=jnp.float32)
        + kb1_ref[...])
    key = (jnp.dot(key, kW2_ref[...], preferred_element_type=jnp.float32)
           + kb2_ref[...]).reshape(EB, _A, _OUT)
    qry = jnp.tanh(
        jnp.dot(hb, qW1_ref[...], preferred_element_type=jnp.float32)
        + qb1_ref[...])
    qry = (jnp.dot(qry, qW2_ref[...], preferred_element_type=jnp.float32)
           + qb2_ref[...]).reshape(EB, _A, _OUT)
    s = jnp.sum(qry[:, :, None, :] * key[:, None, :, :], axis=-1)
    w = jax.nn.sigmoid(s * 0.125)
    pi = pi_ref[...]
    act = act_ref[...]
    pib = pi[:, None, :, :]
    z = w[..., None] * (act[:, None, :, :] - pib) + pib
    S = jnp.sum(z, axis=2)
    zz = (S[:, :, None, :] - z + pib) * 0.125
    obs = obs_ref[...]
    for i in range(_A):
        out_ref[:, i, :, 0:_D] = obs
    out_ref[:, :, :, _D:] = zz
    w_ref[...] = w


def kernel(h, policies, actions, obs_proc, edge_index,
           kW1, kb1, kW2, kb2, qW1, qb1, qW2, qb2):
    N = h.shape[0]
    E = N // _A
    EB = 256
    grid = (E // EB,)
    h3 = h.reshape(E, _A, _D)
    pi3 = policies.reshape(E, _A, _NA)
    act3 = actions.reshape(E, _A, _NA)
    obs3 = obs_proc.reshape(E, _A, _D)

    def blk(shape):
        return pl.BlockSpec(shape, lambda b: (b,) + (0,) * (len(shape) - 1))

    def full(shape):
        return pl.BlockSpec(shape, lambda b: (0,) * len(shape))

    out, w = pl.pallas_call(
        _body,
        grid=grid,
        in_specs=[
            blk((EB, _A, _D)),
            blk((EB, _A, _NA)),
            blk((EB, _A, _NA)),
            blk((EB, _A, _D)),
            full((_D, 32)), full((1, 32)), full((32, _OUT)), full((1, _OUT)),
            full((_D, 32)), full((1, 32)), full((32, _OUT)), full((1, _OUT)),
        ],
        out_specs=[
            blk((EB, _A, _A, _D + _NA)),
            blk((EB, _A, _A)),
        ],
        out_shape=[
            jax.ShapeDtypeStruct((E, _A, _A, _D + _NA), jnp.float32),
            jax.ShapeDtypeStruct((E, _A, _A), jnp.float32),
        ],
    )(h3, pi3, act3, obs3,
      kW1, kb1.reshape(1, 32), kW2, kb2.reshape(1, _OUT),
      qW1, qb1.reshape(1, 32), qW2, qb2.reshape(1, _OUT))
    return out.reshape(N, _A, _D + _NA), w.reshape(N, _A, 1)
